# baseline (speedup 1.0000x reference)
import functools

import jax
import jax.numpy as jnp
from jax import lax
from jax.experimental import pallas as pl
from jax.experimental.pallas import tpu as pltpu
from jax.experimental.pallas import tpu_sc as plsc

N = 10000
E = 320000
G = 256
DIN = 128
H = 64
D3 = 3 * H
STEPS = 3

NC, NS = 2, 16
NW = NC * NS
EPT = E // NW
CSZ = 80
NCH = EPT // CSZ
NP = 10240
RPT = NP // NS

F32 = jnp.float32


def _sc_aggr_build(with_deg):
    outs = [jax.ShapeDtypeStruct((NC, NP, H), F32)]
    scratch = [
        pltpu.VMEM((NCH, CSZ), jnp.int32),
        pltpu.VMEM((NCH, CSZ), jnp.int32),
        pltpu.VMEM((CSZ, H), F32),
        pltpu.VMEM((CSZ, H), F32),
        pltpu.VMEM((CSZ, H), F32),
        pltpu.VMEM((CSZ, H), F32),
        pltpu.VMEM((CSZ, H), F32),
        pltpu.VMEM_SHARED((NP, H), F32),
        pltpu.SemaphoreType.DMA,
        pltpu.SemaphoreType.DMA,
        pltpu.SemaphoreType.DMA,
        pltpu.SemaphoreType.DMA,
        pltpu.SemaphoreType.DMA,
        pltpu.SemaphoreType.DMA,
        pltpu.SemaphoreType.DMA,
        pltpu.SemaphoreType.DMA,
    ]
    if with_deg:
        outs.append(jax.ShapeDtypeStruct((NC, NP, 16), F32))
        scratch += [
            pltpu.VMEM((CSZ, 16), F32),
            pltpu.VMEM((CSZ, 16), F32),
            pltpu.VMEM_SHARED((NP, 16), F32),
            pltpu.SemaphoreType.DMA,
        ]

    def body(hs_hbm, src_hbm, dst_hbm, *rest):
        if with_deg:
            (out_hbm, deg_hbm, src_v, dst_v, r0, r1, r2, r3, zero_v, acc,
             g0, g1, g2, g3, s0, s1, s2, s3,
             zero16_v, ones_v, dacc, dsem) = rest
        else:
            (out_hbm, src_v, dst_v, r0, r1, r2, r3, zero_v, acc,
             g0, g1, g2, g3, s0, s1, s2, s3) = rest
        rows = (r0, r1, r2, r3)
        gsem = (g0, g1, g2, g3)
        ssem = (s0, s1, s2, s3)
        cid = lax.axis_index("c")
        sid = lax.axis_index("s")
        wid = sid * NC + cid
        z16 = jnp.zeros((16,), F32)

        def zrow(i, carry):
            for k in range(H // 16):
                zero_v[i, pl.ds(k * 16, 16)] = z16
            if with_deg:
                zero16_v[i, :] = z16
            return carry
        lax.fori_loop(0, CSZ, zrow, 0)
        if with_deg:
            o16 = jnp.ones((16,), F32)

            def orow(i, carry):
                ones_v[i, :] = o16
                return carry
            lax.fori_loop(0, CSZ, orow, 0)
        row0 = sid * RPT
        for mblk in range(RPT // CSZ):
            pltpu.sync_copy(zero_v, acc.at[pl.ds(row0 + mblk * CSZ, CSZ)])
            if with_deg:
                pltpu.sync_copy(zero16_v,
                                dacc.at[pl.ds(row0 + mblk * CSZ, CSZ)])
        pltpu.sync_copy(src_hbm.at[wid], src_v)
        pltpu.sync_copy(dst_hbm.at[wid], dst_v)
        plsc.subcore_barrier()

        pltpu.async_copy(hs_hbm.at[src_v.at[0]], rows[0], gsem[0])
        pltpu.async_copy(hs_hbm.at[src_v.at[1]], rows[1], gsem[1])

        def slot(j, i):
            b = (i + 2) % 4
            pltpu.make_async_copy(hs_hbm.at[src_v.at[j]], rows[i],
                                  gsem[i]).wait()
            pltpu.async_copy(rows[i], acc.at[dst_v.at[j]], ssem[i], add=True)
            if with_deg:
                @pl.when(j >= 1)
                def _():
                    pltpu.make_async_copy(ones_v, dacc.at[dst_v.at[j - 1]],
                                          dsem).wait()
                pltpu.async_copy(ones_v, dacc.at[dst_v.at[j]], dsem, add=True)

            @pl.when(jnp.logical_and(j >= 2, j + 2 < NCH))
            def _():
                pltpu.make_async_copy(rows[b], acc.at[dst_v.at[j - 2]],
                                      ssem[b]).wait()

            @pl.when(j + 2 < NCH)
            def _():
                pltpu.async_copy(hs_hbm.at[src_v.at[j + 2]], rows[b], gsem[b])

        def group(k, carry):
            for i in range(4):
                slot(4 * k + i, i)
            return carry
        lax.fori_loop(0, NCH // 4, group, 0)
        for j in range(4 * (NCH // 4), NCH):
            slot(j, j % 4)
        for j in range(NCH - 4, NCH):
            pltpu.make_async_copy(rows[j % 4], acc.at[dst_v.at[j]],
                                  ssem[j % 4]).wait()
        if with_deg:
            pltpu.make_async_copy(ones_v, dacc.at[dst_v.at[NCH - 1]],
                                  dsem).wait()

        plsc.subcore_barrier()
        pltpu.sync_copy(acc.at[pl.ds(row0, RPT)],
                        out_hbm.at[cid, pl.ds(row0, RPT)])
        if with_deg:
            pltpu.sync_copy(dacc.at[pl.ds(row0, RPT)],
                            deg_hbm.at[cid, pl.ds(row0, RPT)])

    mesh = plsc.VectorSubcoreMesh(core_axis_name="c", subcore_axis_name="s",
                                  num_cores=NC, num_subcores=NS)
    return pl.kernel(body, out_type=tuple(outs) if with_deg else outs[0],
                     mesh=mesh, scratch_types=scratch,
                     compiler_params=pltpu.CompilerParams(
                         use_tc_tiling_on_sc=False))


@functools.lru_cache(maxsize=None)
def _sc_aggr_get(with_deg):
    return _sc_aggr_build(with_deg)


def _mmT(a, b):
    return lax.dot_general(a, b, (((1,), (1,)), ((), ())),
                           preferred_element_type=F32)


BN = 1000
NB = N // BN


def _proj_body(x_ref, w_ref, b_ref, o_ref):
    o_ref[...] = jnp.maximum(_mmT(x_ref[...], w_ref[...]) + b_ref[...], 0.0)


_proj = pl.pallas_call(
    _proj_body,
    grid=(NB,),
    in_specs=[
        pl.BlockSpec((BN, DIN), lambda i: (i, 0)),
        pl.BlockSpec((H, DIN), lambda i: (0, 0)),
        pl.BlockSpec((1, H), lambda i: (0, 0)),
    ],
    out_specs=pl.BlockSpec((BN, H), lambda i: (i, 0)),
    out_shape=jax.ShapeDtypeStruct((N, H), F32),
)


def _gru_combine(gi, gh, h):
    r = jax.nn.sigmoid(gi[:, 0:H] + gh[:, 0:H])
    z = jax.nn.sigmoid(gi[:, H:2 * H] + gh[:, H:2 * H])
    n = jnp.tanh(gi[:, 2 * H:3 * H] + r * gh[:, 2 * H:3 * H])
    return (1.0 - z) * n + z * h


def _step_body(p_ref, d_ref, hs_ref, hr_ref, wl_ref, bl_ref, wr_ref,
               wgs_ref, bgs_ref, wgr_ref, bgr_ref,
               wihs_ref, whhs_ref, bihs_ref, bhhs_ref,
               wihr_ref, whhr_ref, bihr_ref, bhhr_ref,
               hs_out, hr_out, gate_out):
    p = p_ref[...]
    d = d_ref[...]
    deg = jnp.maximum(d[0, :, 0:1] + d[1, :, 0:1], 1.0)
    aggr = (p[0] + p[1]) / deg
    hs = hs_ref[...]
    hr = hr_ref[...]
    am = jnp.maximum(
        _mmT(aggr, wl_ref[...]) + bl_ref[...] + _mmT(hs, wr_ref[...]), 0.0)
    gate_s = jax.nn.sigmoid(
        jnp.sum(hs * wgs_ref[...], axis=1, keepdims=True) + bgs_ref[0, 0])
    gi_s = _mmT(gate_s * am, wihs_ref[...]) + bihs_ref[...]
    gh_s = _mmT(hs, whhs_ref[...]) + bhhs_ref[...]
    hs_out[...] = _gru_combine(gi_s, gh_s, hs)
    gate_r = jax.nn.sigmoid(
        jnp.sum(hr * wgr_ref[...], axis=1, keepdims=True) + bgr_ref[0, 0])
    gi_r = _mmT(gate_r * am, wihr_ref[...]) + bihr_ref[...]
    gh_r = _mmT(hr, whhr_ref[...]) + bhhr_ref[...]
    hr_out[...] = _gru_combine(gi_r, gh_r, hr)
    gate_out[...] = gate_r


def _full(shape):
    return pl.BlockSpec(shape, lambda i: tuple(0 for _ in shape))


def _rows(w):
    return pl.BlockSpec((BN, w), lambda i: (i, 0))


_step = pl.pallas_call(
    _step_body,
    grid=(NB,),
    in_specs=[
        pl.BlockSpec((NC, BN, H), lambda i: (0, i, 0)),
        pl.BlockSpec((NC, BN, 16), lambda i: (0, i, 0)),
        _rows(H), _rows(H),
        _full((H, H)), _full((1, H)), _full((H, H)),
        _full((1, H)), _full((1, 1)), _full((1, H)), _full((1, 1)),
        _full((3 * H, H)), _full((3 * H, H)), _full((1, 3 * H)),
        _full((1, 3 * H)),
        _full((3 * H, H)), _full((3 * H, H)), _full((1, 3 * H)),
        _full((1, 3 * H)),
    ],
    out_specs=[_rows(H), _rows(H), _rows(1)],
    out_shape=[
        jax.ShapeDtypeStruct((N, H), F32),
        jax.ShapeDtypeStruct((N, H), F32),
        jax.ShapeDtypeStruct((N, 1), F32),
    ],
)

CB = 400
NCK = N // CB
NEG = -1e30


def _s2s_body(h0_ref, hs_ref, hr_ref, b_ref, wih_ref, whh_ref, bih_ref,
              bhh_ref, wsp_ref, bsp_ref, ap_ref, out_ref, na_ref, ea_ref):
    na_ref[:, 0:H] = h0_ref[...]
    na_ref[:, H:2 * H] = hs_ref[...]
    na_ref[:, 2 * H:3 * H] = hr_ref[...]
    wih = wih_ref[...]
    whh = whh_ref[...]
    bih = bih_ref[...]
    bhh = bhh_ref[...]
    h = jnp.zeros((G, D3), F32)
    c = jnp.zeros((G, D3), F32)
    q_star = jnp.zeros((G, 2 * D3), F32)
    for _ in range(STEPS):
        g = _mmT(q_star, wih) + bih + _mmT(h, whh) + bhh
        i_ = jax.nn.sigmoid(g[:, 0:D3])
        f_ = jax.nn.sigmoid(g[:, D3:2 * D3])
        gg = jnp.tanh(g[:, 2 * D3:3 * D3])
        o_ = jax.nn.sigmoid(g[:, 3 * D3:4 * D3])
        c = f_ * c + i_ * gg
        h = o_ * jnp.tanh(c)
        q = h

        def ph1(ib, mcar):
            sl = pl.ds(ib * CB, CB)
            bc = b_ref[sl, :]
            oh = (bc == lax.broadcasted_iota(jnp.int32, (CB, G), 1)).astype(F32)
            qn = lax.dot_general(oh, q, (((1,), (0,)), ((), ())),
                                 preferred_element_type=F32)
            e = jnp.sum(na_ref[sl, :] * qn, axis=1, keepdims=True)
            ea_ref[sl, :] = e
            masked = jnp.where(oh > 0.5, e, NEG)
            return jnp.maximum(mcar, jnp.max(masked, axis=0, keepdims=True))
        m = lax.fori_loop(0, NCK, ph1, jnp.full((1, G), NEG, F32))

        def ph2(ib, scar):
            sl = pl.ds(ib * CB, CB)
            bc = b_ref[sl, :]
            oh = (bc == lax.broadcasted_iota(jnp.int32, (CB, G), 1)).astype(F32)
            mn = jnp.sum(oh * m, axis=1, keepdims=True)
            ex = jnp.exp(ea_ref[sl, :] - mn)
            ea_ref[sl, :] = ex
            return scar + lax.dot_general(ex, oh, (((0,), (0,)), ((), ())),
                                          preferred_element_type=F32)
        s = lax.fori_loop(0, NCK, ph2, jnp.zeros((1, G), F32))

        def ph3(ib, rcar):
            sl = pl.ds(ib * CB, CB)
            bc = b_ref[sl, :]
            oh = (bc == lax.broadcasted_iota(jnp.int32, (CB, G), 1)).astype(F32)
            sn = jnp.sum(oh * s, axis=1, keepdims=True)
            a = ea_ref[sl, :] / (sn + 1e-16)
            wna = a * na_ref[sl, :]
            return rcar + lax.dot_general(oh, wna, (((0,), (0,)), ((), ())),
                                          preferred_element_type=F32)
        r = lax.fori_loop(0, NCK, ph3, jnp.zeros((G, D3), F32))
        q_star = jnp.concatenate([q, r], axis=1)
    out = _mmT(q_star, wsp_ref[...]) + bsp_ref[...]
    ap = ap_ref[0, 0]
    out_ref[...] = jnp.maximum(out, 0.0) + ap * jnp.minimum(out, 0.0)


_s2s = pl.pallas_call(
    _s2s_body,
    out_shape=jax.ShapeDtypeStruct((G, 1024), F32),
    scratch_shapes=[
        pltpu.VMEM((N, D3), F32),
        pltpu.VMEM((N, 1), F32),
    ],
)


def kernel(x, edge_index, batch, W_proj, b_proj, W_l, b_l, W_r, W_gs, b_gs,
           W_gr, b_gr, Wih_s, Whh_s, bih_s, bhh_s, Wih_r, Whh_r, bih_r, bhh_r,
           Wih_l, Whh_l, bih_l, bhh_l, W_sp, b_sp, a_prelu):
    src = edge_index[0].astype(jnp.int32).reshape(NW, NCH, CSZ)
    dst = edge_index[1].astype(jnp.int32).reshape(NW, NCH, CSZ)
    b2 = batch.astype(jnp.int32).reshape(N, 1)
    h0 = _proj(x, W_proj, b_proj.reshape(1, H))
    parts0, degp = _sc_aggr_get(True)(h0, src, dst)
    hs, hr = h0, h0
    gates = []
    step_w = (W_l, b_l.reshape(1, H), W_r, W_gs, b_gs.reshape(1, 1),
              W_gr, b_gr.reshape(1, 1),
              Wih_s, Whh_s, bih_s.reshape(1, 3 * H), bhh_s.reshape(1, 3 * H),
              Wih_r, Whh_r, bih_r.reshape(1, 3 * H), bhh_r.reshape(1, 3 * H))
    for s in range(STEPS):
        parts = parts0 if s == 0 else _sc_aggr_get(False)(hs, src, dst)
        hs, hr, gr = _step(parts, degp, hs, hr, *step_w)
        gates.append(gr)
    out = _s2s(h0, hs, hr, b2, Wih_l, Whh_l, bih_l.reshape(1, 4 * D3),
               bhh_l.reshape(1, 4 * D3), W_sp, b_sp.reshape(1, 1024),
               a_prelu.reshape(1, 1))
    return out, jnp.stack(gates, axis=0)


# speedup vs baseline: 1.1056x; 1.1056x over previous
import functools

import jax
import jax.numpy as jnp
from jax import lax
from jax.experimental import pallas as pl
from jax.experimental.pallas import tpu as pltpu
from jax.experimental.pallas import tpu_sc as plsc

N = 10000
E = 320000
G = 256
DIN = 128
H = 64
D3 = 3 * H
STEPS = 3

NC, NS = 2, 16
NW = NC * NS
EPT = E // NW
CSZ = 80
NCH = EPT // CSZ
NP = 10240
RPT = NP // NS

F32 = jnp.float32


def _sc_aggr_build(with_deg):
    outs = [jax.ShapeDtypeStruct((NC, NP, H), F32)]
    scratch = [
        pltpu.VMEM((NCH, CSZ), jnp.int32),
        pltpu.VMEM((NCH, CSZ), jnp.int32),
        pltpu.VMEM((CSZ, H), F32),
        pltpu.VMEM((CSZ, H), F32),
        pltpu.VMEM((CSZ, H), F32),
        pltpu.VMEM((CSZ, H), F32),
        pltpu.VMEM((CSZ, H), F32),
        pltpu.VMEM_SHARED((NP, H), F32),
        pltpu.SemaphoreType.DMA,
        pltpu.SemaphoreType.DMA,
        pltpu.SemaphoreType.DMA,
        pltpu.SemaphoreType.DMA,
        pltpu.SemaphoreType.DMA,
        pltpu.SemaphoreType.DMA,
        pltpu.SemaphoreType.DMA,
        pltpu.SemaphoreType.DMA,
    ]
    if with_deg:
        outs.append(jax.ShapeDtypeStruct((NC, NP, 16), F32))
        scratch += [
            pltpu.VMEM((CSZ, 16), F32),
            pltpu.VMEM((CSZ, 16), F32),
            pltpu.VMEM_SHARED((NP, 16), F32),
            pltpu.SemaphoreType.DMA,
        ]

    def body(hs_hbm, src_hbm, dst_hbm, *rest):
        if with_deg:
            (out_hbm, deg_hbm, src_v, dst_v, r0, r1, r2, r3, zero_v, acc,
             g0, g1, g2, g3, s0, s1, s2, s3,
             zero16_v, ones_v, dacc, dsem) = rest
        else:
            (out_hbm, src_v, dst_v, r0, r1, r2, r3, zero_v, acc,
             g0, g1, g2, g3, s0, s1, s2, s3) = rest
        rows = (r0, r1, r2, r3)
        gsem = (g0, g1, g2, g3)
        ssem = (s0, s1, s2, s3)
        cid = lax.axis_index("c")
        sid = lax.axis_index("s")
        wid = sid * NC + cid
        z16 = jnp.zeros((16,), F32)

        def zrow(i, carry):
            for k in range(H // 16):
                zero_v[i, pl.ds(k * 16, 16)] = z16
            if with_deg:
                zero16_v[i, :] = z16
            return carry
        lax.fori_loop(0, CSZ, zrow, 0)
        if with_deg:
            o16 = jnp.ones((16,), F32)

            def orow(i, carry):
                ones_v[i, :] = o16
                return carry
            lax.fori_loop(0, CSZ, orow, 0)
        row0 = sid * RPT
        for mblk in range(RPT // CSZ):
            pltpu.sync_copy(zero_v, acc.at[pl.ds(row0 + mblk * CSZ, CSZ)])
            if with_deg:
                pltpu.sync_copy(zero16_v,
                                dacc.at[pl.ds(row0 + mblk * CSZ, CSZ)])
        pltpu.sync_copy(src_hbm.at[wid], src_v)
        pltpu.sync_copy(dst_hbm.at[wid], dst_v)
        plsc.subcore_barrier()

        pltpu.async_copy(hs_hbm.at[src_v.at[0]], rows[0], gsem[0])
        pltpu.async_copy(hs_hbm.at[src_v.at[1]], rows[1], gsem[1])

        def slot(j, i):
            b = (i + 2) % 4
            pltpu.make_async_copy(hs_hbm.at[src_v.at[j]], rows[i],
                                  gsem[i]).wait()
            pltpu.async_copy(rows[i], acc.at[dst_v.at[j]], ssem[i], add=True)
            if with_deg:
                @pl.when(j >= 1)
                def _():
                    pltpu.make_async_copy(ones_v, dacc.at[dst_v.at[j - 1]],
                                          dsem).wait()
                pltpu.async_copy(ones_v, dacc.at[dst_v.at[j]], dsem, add=True)

            @pl.when(jnp.logical_and(j >= 2, j + 2 < NCH))
            def _():
                pltpu.make_async_copy(rows[b], acc.at[dst_v.at[j - 2]],
                                      ssem[b]).wait()

            @pl.when(j + 2 < NCH)
            def _():
                pltpu.async_copy(hs_hbm.at[src_v.at[j + 2]], rows[b], gsem[b])

        def group(k, carry):
            for i in range(4):
                slot(4 * k + i, i)
            return carry
        lax.fori_loop(0, NCH // 4, group, 0)
        for j in range(4 * (NCH // 4), NCH):
            slot(j, j % 4)
        for j in range(NCH - 4, NCH):
            pltpu.make_async_copy(rows[j % 4], acc.at[dst_v.at[j]],
                                  ssem[j % 4]).wait()
        if with_deg:
            pltpu.make_async_copy(ones_v, dacc.at[dst_v.at[NCH - 1]],
                                  dsem).wait()

        plsc.subcore_barrier()
        pltpu.sync_copy(acc.at[pl.ds(row0, RPT)],
                        out_hbm.at[cid, pl.ds(row0, RPT)])
        if with_deg:
            pltpu.sync_copy(dacc.at[pl.ds(row0, RPT)],
                            deg_hbm.at[cid, pl.ds(row0, RPT)])

    mesh = plsc.VectorSubcoreMesh(core_axis_name="c", subcore_axis_name="s",
                                  num_cores=NC, num_subcores=NS)
    return pl.kernel(body, out_type=tuple(outs) if with_deg else outs[0],
                     mesh=mesh, scratch_types=scratch,
                     compiler_params=pltpu.CompilerParams(
                         use_tc_tiling_on_sc=False))


@functools.lru_cache(maxsize=None)
def _sc_aggr_get(with_deg):
    return _sc_aggr_build(with_deg)


def _mmT(a, b):
    return lax.dot_general(a, b, (((1,), (1,)), ((), ())),
                           preferred_element_type=F32)


BN = 1000
NB = N // BN


def _proj_body(x_ref, w_ref, b_ref, o_ref):
    o_ref[...] = jnp.maximum(_mmT(x_ref[...], w_ref[...]) + b_ref[...], 0.0)


_proj = pl.pallas_call(
    _proj_body,
    grid=(NB,),
    in_specs=[
        pl.BlockSpec((BN, DIN), lambda i: (i, 0)),
        pl.BlockSpec((H, DIN), lambda i: (0, 0)),
        pl.BlockSpec((1, H), lambda i: (0, 0)),
    ],
    out_specs=pl.BlockSpec((BN, H), lambda i: (i, 0)),
    out_shape=jax.ShapeDtypeStruct((N, H), F32),
)


def _gru_combine(gi, gh, h):
    r = jax.nn.sigmoid(gi[:, 0:H] + gh[:, 0:H])
    z = jax.nn.sigmoid(gi[:, H:2 * H] + gh[:, H:2 * H])
    n = jnp.tanh(gi[:, 2 * H:3 * H] + r * gh[:, 2 * H:3 * H])
    return (1.0 - z) * n + z * h


def _step_body(p_ref, d_ref, hs_ref, hr_ref, wl_ref, bl_ref, wr_ref,
               wgs_ref, bgs_ref, wgr_ref, bgr_ref,
               wihs_ref, whhs_ref, bihs_ref, bhhs_ref,
               wihr_ref, whhr_ref, bihr_ref, bhhr_ref,
               hs_out, hr_out, gate_out):
    p = p_ref[...]
    d = d_ref[...]
    deg = jnp.maximum(d[0, :, 0:1] + d[1, :, 0:1], 1.0)
    aggr = (p[0] + p[1]) / deg
    hs = hs_ref[...]
    hr = hr_ref[...]
    am = jnp.maximum(
        _mmT(aggr, wl_ref[...]) + bl_ref[...] + _mmT(hs, wr_ref[...]), 0.0)
    gate_s = jax.nn.sigmoid(
        jnp.sum(hs * wgs_ref[...], axis=1, keepdims=True) + bgs_ref[0, 0])
    gi_s = _mmT(gate_s * am, wihs_ref[...]) + bihs_ref[...]
    gh_s = _mmT(hs, whhs_ref[...]) + bhhs_ref[...]
    hs_out[...] = _gru_combine(gi_s, gh_s, hs)
    gate_r = jax.nn.sigmoid(
        jnp.sum(hr * wgr_ref[...], axis=1, keepdims=True) + bgr_ref[0, 0])
    gi_r = _mmT(gate_r * am, wihr_ref[...]) + bihr_ref[...]
    gh_r = _mmT(hr, whhr_ref[...]) + bhhr_ref[...]
    hr_out[...] = _gru_combine(gi_r, gh_r, hr)
    gate_out[...] = gate_r


def _full(shape):
    return pl.BlockSpec(shape, lambda i: tuple(0 for _ in shape))


def _rows(w):
    return pl.BlockSpec((BN, w), lambda i: (i, 0))


_step = pl.pallas_call(
    _step_body,
    grid=(NB,),
    in_specs=[
        pl.BlockSpec((NC, BN, H), lambda i: (0, i, 0)),
        pl.BlockSpec((NC, BN, 16), lambda i: (0, i, 0)),
        _rows(H), _rows(H),
        _full((H, H)), _full((1, H)), _full((H, H)),
        _full((1, H)), _full((1, 1)), _full((1, H)), _full((1, 1)),
        _full((3 * H, H)), _full((3 * H, H)), _full((1, 3 * H)),
        _full((1, 3 * H)),
        _full((3 * H, H)), _full((3 * H, H)), _full((1, 3 * H)),
        _full((1, 3 * H)),
    ],
    out_specs=[_rows(H), _rows(H), _rows(1)],
    out_shape=[
        jax.ShapeDtypeStruct((N, H), F32),
        jax.ShapeDtypeStruct((N, H), F32),
        jax.ShapeDtypeStruct((N, 1), F32),
    ],
)

CB = 1000
NCK = N // CB
NEG = -1e30


def _s2s_body(h0_ref, hs_ref, hr_ref, b_ref, wih_ref, whh_ref, bih_ref,
              bhh_ref, wsp_ref, bsp_ref, ap_ref, out_ref, na_ref, ea_ref):
    na_ref[:, 0:H] = h0_ref[...]
    na_ref[:, H:2 * H] = hs_ref[...]
    na_ref[:, 2 * H:3 * H] = hr_ref[...]
    wih = wih_ref[...]
    whh = whh_ref[...]
    bih = bih_ref[...]
    bhh = bhh_ref[...]
    h = jnp.zeros((G, D3), F32)
    c = jnp.zeros((G, D3), F32)
    q_star = jnp.zeros((G, 2 * D3), F32)
    for _ in range(STEPS):
        g = _mmT(q_star, wih) + bih + _mmT(h, whh) + bhh
        i_ = jax.nn.sigmoid(g[:, 0:D3])
        f_ = jax.nn.sigmoid(g[:, D3:2 * D3])
        gg = jnp.tanh(g[:, 2 * D3:3 * D3])
        o_ = jax.nn.sigmoid(g[:, 3 * D3:4 * D3])
        c = f_ * c + i_ * gg
        h = o_ * jnp.tanh(c)
        q = h

        def ph1(ib, mcar):
            sl = pl.ds(ib * CB, CB)
            bc = b_ref[sl, :]
            oh = (bc == lax.broadcasted_iota(jnp.int32, (CB, G), 1)).astype(F32)
            qn = lax.dot_general(oh, q, (((1,), (0,)), ((), ())),
                                 preferred_element_type=F32)
            e = jnp.sum(na_ref[sl, :] * qn, axis=1, keepdims=True)
            ea_ref[sl, :] = e
            masked = jnp.where(oh > 0.5, e, NEG)
            return jnp.maximum(mcar, jnp.max(masked, axis=0, keepdims=True))
        m = lax.fori_loop(0, NCK, ph1, jnp.full((1, G), NEG, F32))

        def ph2(ib, scar):
            sl = pl.ds(ib * CB, CB)
            bc = b_ref[sl, :]
            oh = (bc == lax.broadcasted_iota(jnp.int32, (CB, G), 1)).astype(F32)
            mn = jnp.sum(oh * m, axis=1, keepdims=True)
            ex = jnp.exp(ea_ref[sl, :] - mn)
            ea_ref[sl, :] = ex
            return scar + lax.dot_general(ex, oh, (((0,), (0,)), ((), ())),
                                          preferred_element_type=F32)
        s = lax.fori_loop(0, NCK, ph2, jnp.zeros((1, G), F32))

        def ph3(ib, rcar):
            sl = pl.ds(ib * CB, CB)
            bc = b_ref[sl, :]
            oh = (bc == lax.broadcasted_iota(jnp.int32, (CB, G), 1)).astype(F32)
            sn = jnp.sum(oh * s, axis=1, keepdims=True)
            a = ea_ref[sl, :] / (sn + 1e-16)
            wna = a * na_ref[sl, :]
            return rcar + lax.dot_general(oh, wna, (((0,), (0,)), ((), ())),
                                          preferred_element_type=F32)
        r = lax.fori_loop(0, NCK, ph3, jnp.zeros((G, D3), F32))
        q_star = jnp.concatenate([q, r], axis=1)
    out = _mmT(q_star, wsp_ref[...]) + bsp_ref[...]
    ap = ap_ref[0, 0]
    out_ref[...] = jnp.maximum(out, 0.0) + ap * jnp.minimum(out, 0.0)


_s2s = pl.pallas_call(
    _s2s_body,
    out_shape=jax.ShapeDtypeStruct((G, 1024), F32),
    scratch_shapes=[
        pltpu.VMEM((N, D3), F32),
        pltpu.VMEM((N, 1), F32),
    ],
)


def kernel(x, edge_index, batch, W_proj, b_proj, W_l, b_l, W_r, W_gs, b_gs,
           W_gr, b_gr, Wih_s, Whh_s, bih_s, bhh_s, Wih_r, Whh_r, bih_r, bhh_r,
           Wih_l, Whh_l, bih_l, bhh_l, W_sp, b_sp, a_prelu):
    src = edge_index[0].astype(jnp.int32).reshape(NW, NCH, CSZ)
    dst = edge_index[1].astype(jnp.int32).reshape(NW, NCH, CSZ)
    b2 = batch.astype(jnp.int32).reshape(N, 1)
    h0 = _proj(x, W_proj, b_proj.reshape(1, H))
    parts0, degp = _sc_aggr_get(True)(h0, src, dst)
    hs, hr = h0, h0
    gates = []
    step_w = (W_l, b_l.reshape(1, H), W_r, W_gs, b_gs.reshape(1, 1),
              W_gr, b_gr.reshape(1, 1),
              Wih_s, Whh_s, bih_s.reshape(1, 3 * H), bhh_s.reshape(1, 3 * H),
              Wih_r, Whh_r, bih_r.reshape(1, 3 * H), bhh_r.reshape(1, 3 * H))
    for s in range(STEPS):
        parts = parts0 if s == 0 else _sc_aggr_get(False)(hs, src, dst)
        hs, hr, gr = _step(parts, degp, hs, hr, *step_w)
        gates.append(gr)
    out = _s2s(h0, hs, hr, b2, Wih_l, Whh_l, bih_l.reshape(1, 4 * D3),
               bhh_l.reshape(1, 4 * D3), W_sp, b_sp.reshape(1, 1024),
               a_prelu.reshape(1, 1))
    return out, jnp.stack(gates, axis=0)


# speedup vs baseline: 1.1404x; 1.0315x over previous
import functools

import jax
import jax.numpy as jnp
from jax import lax
from jax.experimental import pallas as pl
from jax.experimental.pallas import tpu as pltpu
from jax.experimental.pallas import tpu_sc as plsc

N = 10000
E = 320000
G = 256
DIN = 128
H = 64
D3 = 3 * H
STEPS = 3

NC, NS = 2, 16
NW = NC * NS
EPT = E // NW
CSZ = 80
NCH = EPT // CSZ
NP = 10240
RPT = NP // NS

F32 = jnp.float32


def _sc_aggr_build(with_deg):
    outs = [jax.ShapeDtypeStruct((NC, NP, H), F32)]
    scratch = [
        pltpu.VMEM((NCH, CSZ), jnp.int32),
        pltpu.VMEM((NCH, CSZ), jnp.int32),
        pltpu.VMEM((CSZ, H), F32),
        pltpu.VMEM((CSZ, H), F32),
        pltpu.VMEM((CSZ, H), F32),
        pltpu.VMEM((CSZ, H), F32),
        pltpu.VMEM((CSZ, H), F32),
        pltpu.VMEM_SHARED((NP, H), F32),
        pltpu.SemaphoreType.DMA,
        pltpu.SemaphoreType.DMA,
        pltpu.SemaphoreType.DMA,
        pltpu.SemaphoreType.DMA,
        pltpu.SemaphoreType.DMA,
        pltpu.SemaphoreType.DMA,
        pltpu.SemaphoreType.DMA,
        pltpu.SemaphoreType.DMA,
    ]
    if with_deg:
        outs.append(jax.ShapeDtypeStruct((NC, NP, 16), F32))
        scratch += [
            pltpu.VMEM((CSZ, 16), F32),
            pltpu.VMEM((CSZ, 16), F32),
            pltpu.VMEM_SHARED((NP, 16), F32),
            pltpu.SemaphoreType.DMA,
        ]

    def body(hs_hbm, src_hbm, dst_hbm, *rest):
        if with_deg:
            (out_hbm, deg_hbm, src_v, dst_v, r0, r1, r2, r3, zero_v, acc,
             g0, g1, g2, g3, s0, s1, s2, s3,
             zero16_v, ones_v, dacc, dsem) = rest
        else:
            (out_hbm, src_v, dst_v, r0, r1, r2, r3, zero_v, acc,
             g0, g1, g2, g3, s0, s1, s2, s3) = rest
        rows = (r0, r1, r2, r3)
        gsem = (g0, g1, g2, g3)
        ssem = (s0, s1, s2, s3)
        cid = lax.axis_index("c")
        sid = lax.axis_index("s")
        wid = sid * NC + cid
        z16 = jnp.zeros((16,), F32)

        def zrow(i, carry):
            for k in range(H // 16):
                zero_v[i, pl.ds(k * 16, 16)] = z16
            if with_deg:
                zero16_v[i, :] = z16
            return carry
        lax.fori_loop(0, CSZ, zrow, 0)
        if with_deg:
            o16 = jnp.ones((16,), F32)

            def orow(i, carry):
                ones_v[i, :] = o16
                return carry
            lax.fori_loop(0, CSZ, orow, 0)
        row0 = sid * RPT
        for mblk in range(RPT // CSZ):
            pltpu.sync_copy(zero_v, acc.at[pl.ds(row0 + mblk * CSZ, CSZ)])
            if with_deg:
                pltpu.sync_copy(zero16_v,
                                dacc.at[pl.ds(row0 + mblk * CSZ, CSZ)])
        pltpu.sync_copy(src_hbm.at[wid], src_v)
        pltpu.sync_copy(dst_hbm.at[wid], dst_v)
        plsc.subcore_barrier()

        pltpu.async_copy(hs_hbm.at[src_v.at[0]], rows[0], gsem[0])
        pltpu.async_copy(hs_hbm.at[src_v.at[1]], rows[1], gsem[1])

        def slot(j, i):
            b = (i + 2) % 4
            pltpu.make_async_copy(hs_hbm.at[src_v.at[j]], rows[i],
                                  gsem[i]).wait()
            pltpu.async_copy(rows[i], acc.at[dst_v.at[j]], ssem[i], add=True)
            if with_deg:
                @pl.when(j >= 1)
                def _():
                    pltpu.make_async_copy(ones_v, dacc.at[dst_v.at[j - 1]],
                                          dsem).wait()
                pltpu.async_copy(ones_v, dacc.at[dst_v.at[j]], dsem, add=True)

            @pl.when(jnp.logical_and(j >= 2, j + 2 < NCH))
            def _():
                pltpu.make_async_copy(rows[b], acc.at[dst_v.at[j - 2]],
                                      ssem[b]).wait()

            @pl.when(j + 2 < NCH)
            def _():
                pltpu.async_copy(hs_hbm.at[src_v.at[j + 2]], rows[b], gsem[b])

        def group(k, carry):
            for i in range(4):
                slot(4 * k + i, i)
            return carry
        lax.fori_loop(0, NCH // 4, group, 0)
        for j in range(4 * (NCH // 4), NCH):
            slot(j, j % 4)
        for j in range(NCH - 4, NCH):
            pltpu.make_async_copy(rows[j % 4], acc.at[dst_v.at[j]],
                                  ssem[j % 4]).wait()
        if with_deg:
            pltpu.make_async_copy(ones_v, dacc.at[dst_v.at[NCH - 1]],
                                  dsem).wait()

        plsc.subcore_barrier()
        pltpu.sync_copy(acc.at[pl.ds(row0, RPT)],
                        out_hbm.at[cid, pl.ds(row0, RPT)])
        if with_deg:
            pltpu.sync_copy(dacc.at[pl.ds(row0, RPT)],
                            deg_hbm.at[cid, pl.ds(row0, RPT)])

    mesh = plsc.VectorSubcoreMesh(core_axis_name="c", subcore_axis_name="s",
                                  num_cores=NC, num_subcores=NS)
    return pl.kernel(body, out_type=tuple(outs) if with_deg else outs[0],
                     mesh=mesh, scratch_types=scratch,
                     compiler_params=pltpu.CompilerParams(
                         use_tc_tiling_on_sc=False))


@functools.lru_cache(maxsize=None)
def _sc_aggr_get(with_deg):
    return _sc_aggr_build(with_deg)


def _mmT(a, b):
    return lax.dot_general(a, b, (((1,), (1,)), ((), ())),
                           preferred_element_type=F32)


BN = 2000
NB = N // BN


def _proj_body(x_ref, w_ref, b_ref, o_ref):
    o_ref[...] = jnp.maximum(_mmT(x_ref[...], w_ref[...]) + b_ref[...], 0.0)


_proj = pl.pallas_call(
    _proj_body,
    grid=(NB,),
    in_specs=[
        pl.BlockSpec((BN, DIN), lambda i: (i, 0)),
        pl.BlockSpec((H, DIN), lambda i: (0, 0)),
        pl.BlockSpec((1, H), lambda i: (0, 0)),
    ],
    out_specs=pl.BlockSpec((BN, H), lambda i: (i, 0)),
    out_shape=jax.ShapeDtypeStruct((N, H), F32),
)


def _gru_combine(gi, gh, h):
    r = jax.nn.sigmoid(gi[:, 0:H] + gh[:, 0:H])
    z = jax.nn.sigmoid(gi[:, H:2 * H] + gh[:, H:2 * H])
    n = jnp.tanh(gi[:, 2 * H:3 * H] + r * gh[:, 2 * H:3 * H])
    return (1.0 - z) * n + z * h


def _step_body(p_ref, d_ref, hs_ref, hr_ref, wl_ref, bl_ref, wr_ref,
               wgs_ref, bgs_ref, wgr_ref, bgr_ref,
               wihs_ref, whhs_ref, bihs_ref, bhhs_ref,
               wihr_ref, whhr_ref, bihr_ref, bhhr_ref,
               hs_out, hr_out, gate_out):
    p = p_ref[...]
    d = d_ref[...]
    deg = jnp.maximum(d[0, :, 0:1] + d[1, :, 0:1], 1.0)
    aggr = (p[0] + p[1]) / deg
    hs = hs_ref[...]
    hr = hr_ref[...]
    am = jnp.maximum(
        _mmT(aggr, wl_ref[...]) + bl_ref[...] + _mmT(hs, wr_ref[...]), 0.0)
    gate_s = jax.nn.sigmoid(
        jnp.sum(hs * wgs_ref[...], axis=1, keepdims=True) + bgs_ref[0, 0])
    gi_s = _mmT(gate_s * am, wihs_ref[...]) + bihs_ref[...]
    gh_s = _mmT(hs, whhs_ref[...]) + bhhs_ref[...]
    hs_out[...] = _gru_combine(gi_s, gh_s, hs)
    gate_r = jax.nn.sigmoid(
        jnp.sum(hr * wgr_ref[...], axis=1, keepdims=True) + bgr_ref[0, 0])
    gi_r = _mmT(gate_r * am, wihr_ref[...]) + bihr_ref[...]
    gh_r = _mmT(hr, whhr_ref[...]) + bhhr_ref[...]
    hr_out[...] = _gru_combine(gi_r, gh_r, hr)
    gate_out[...] = gate_r


def _full(shape):
    return pl.BlockSpec(shape, lambda i: tuple(0 for _ in shape))


def _rows(w):
    return pl.BlockSpec((BN, w), lambda i: (i, 0))


_step = pl.pallas_call(
    _step_body,
    grid=(NB,),
    in_specs=[
        pl.BlockSpec((NC, BN, H), lambda i: (0, i, 0)),
        pl.BlockSpec((NC, BN, 16), lambda i: (0, i, 0)),
        _rows(H), _rows(H),
        _full((H, H)), _full((1, H)), _full((H, H)),
        _full((1, H)), _full((1, 1)), _full((1, H)), _full((1, 1)),
        _full((3 * H, H)), _full((3 * H, H)), _full((1, 3 * H)),
        _full((1, 3 * H)),
        _full((3 * H, H)), _full((3 * H, H)), _full((1, 3 * H)),
        _full((1, 3 * H)),
    ],
    out_specs=[_rows(H), _rows(H), _rows(1)],
    out_shape=[
        jax.ShapeDtypeStruct((N, H), F32),
        jax.ShapeDtypeStruct((N, H), F32),
        jax.ShapeDtypeStruct((N, 1), F32),
    ],
)

CB = 2000
NCK = N // CB
NEG = -1e30


def _s2s_body(h0_ref, hs_ref, hr_ref, b_ref, wih_ref, whh_ref, bih_ref,
              bhh_ref, wsp_ref, bsp_ref, ap_ref, out_ref, na_ref, ea_ref):
    na_ref[:, 0:H] = h0_ref[...]
    na_ref[:, H:2 * H] = hs_ref[...]
    na_ref[:, 2 * H:3 * H] = hr_ref[...]
    wih = wih_ref[...]
    whh = whh_ref[...]
    bih = bih_ref[...]
    bhh = bhh_ref[...]
    h = jnp.zeros((G, D3), F32)
    c = jnp.zeros((G, D3), F32)
    q_star = jnp.zeros((G, 2 * D3), F32)
    for _ in range(STEPS):
        g = _mmT(q_star, wih) + bih + _mmT(h, whh) + bhh
        i_ = jax.nn.sigmoid(g[:, 0:D3])
        f_ = jax.nn.sigmoid(g[:, D3:2 * D3])
        gg = jnp.tanh(g[:, 2 * D3:3 * D3])
        o_ = jax.nn.sigmoid(g[:, 3 * D3:4 * D3])
        c = f_ * c + i_ * gg
        h = o_ * jnp.tanh(c)
        q = h

        def ph1(ib, mcar):
            sl = pl.ds(ib * CB, CB)
            bc = b_ref[sl, :]
            oh = (bc == lax.broadcasted_iota(jnp.int32, (CB, G), 1)).astype(F32)
            qn = lax.dot_general(oh, q, (((1,), (0,)), ((), ())),
                                 preferred_element_type=F32)
            e = jnp.sum(na_ref[sl, :] * qn, axis=1, keepdims=True)
            ea_ref[sl, :] = e
            masked = jnp.where(oh > 0.5, e, NEG)
            return jnp.maximum(mcar, jnp.max(masked, axis=0, keepdims=True))
        m = lax.fori_loop(0, NCK, ph1, jnp.full((1, G), NEG, F32))

        def ph2(ib, scar):
            sl = pl.ds(ib * CB, CB)
            bc = b_ref[sl, :]
            oh = (bc == lax.broadcasted_iota(jnp.int32, (CB, G), 1)).astype(F32)
            mn = jnp.sum(oh * m, axis=1, keepdims=True)
            ex = jnp.exp(ea_ref[sl, :] - mn)
            ea_ref[sl, :] = ex
            return scar + lax.dot_general(ex, oh, (((0,), (0,)), ((), ())),
                                          preferred_element_type=F32)
        s = lax.fori_loop(0, NCK, ph2, jnp.zeros((1, G), F32))

        def ph3(ib, rcar):
            sl = pl.ds(ib * CB, CB)
            bc = b_ref[sl, :]
            oh = (bc == lax.broadcasted_iota(jnp.int32, (CB, G), 1)).astype(F32)
            sn = jnp.sum(oh * s, axis=1, keepdims=True)
            a = ea_ref[sl, :] / (sn + 1e-16)
            wna = a * na_ref[sl, :]
            return rcar + lax.dot_general(oh, wna, (((0,), (0,)), ((), ())),
                                          preferred_element_type=F32)
        r = lax.fori_loop(0, NCK, ph3, jnp.zeros((G, D3), F32))
        q_star = jnp.concatenate([q, r], axis=1)
    out = _mmT(q_star, wsp_ref[...]) + bsp_ref[...]
    ap = ap_ref[0, 0]
    out_ref[...] = jnp.maximum(out, 0.0) + ap * jnp.minimum(out, 0.0)


_s2s = pl.pallas_call(
    _s2s_body,
    out_shape=jax.ShapeDtypeStruct((G, 1024), F32),
    scratch_shapes=[
        pltpu.VMEM((N, D3), F32),
        pltpu.VMEM((N, 1), F32),
    ],
)


def kernel(x, edge_index, batch, W_proj, b_proj, W_l, b_l, W_r, W_gs, b_gs,
           W_gr, b_gr, Wih_s, Whh_s, bih_s, bhh_s, Wih_r, Whh_r, bih_r, bhh_r,
           Wih_l, Whh_l, bih_l, bhh_l, W_sp, b_sp, a_prelu):
    src = edge_index[0].astype(jnp.int32).reshape(NW, NCH, CSZ)
    dst = edge_index[1].astype(jnp.int32).reshape(NW, NCH, CSZ)
    b2 = batch.astype(jnp.int32).reshape(N, 1)
    h0 = _proj(x, W_proj, b_proj.reshape(1, H))
    parts0, degp = _sc_aggr_get(True)(h0, src, dst)
    hs, hr = h0, h0
    gates = []
    step_w = (W_l, b_l.reshape(1, H), W_r, W_gs, b_gs.reshape(1, 1),
              W_gr, b_gr.reshape(1, 1),
              Wih_s, Whh_s, bih_s.reshape(1, 3 * H), bhh_s.reshape(1, 3 * H),
              Wih_r, Whh_r, bih_r.reshape(1, 3 * H), bhh_r.reshape(1, 3 * H))
    for s in range(STEPS):
        parts = parts0 if s == 0 else _sc_aggr_get(False)(hs, src, dst)
        hs, hr, gr = _step(parts, degp, hs, hr, *step_w)
        gates.append(gr)
    out = _s2s(h0, hs, hr, b2, Wih_l, Whh_l, bih_l.reshape(1, 4 * D3),
               bhh_l.reshape(1, 4 * D3), W_sp, b_sp.reshape(1, 1024),
               a_prelu.reshape(1, 1))
    return out, jnp.stack(gates, axis=0)
